# baseline (speedup 1.0000x reference)
import dataclasses
import functools

import jax
import jax.numpy as jnp
from jax import lax
from jax.experimental import pallas as pl
from jax.experimental.pallas import tpu as pltpu
from jax.experimental.pallas import tpu_sc as plsc

NUM_CORES = 2
NUM_SUBCORES = 16
NUM_TILES = NUM_CORES * NUM_SUBCORES
LANES = 16

BATCH = 16384
DIM = 64
ROWS_PER_TILE = BATCH // NUM_TILES
CHUNK = 128
NCHUNK = ROWS_PER_TILE // CHUNK
DIM_VREGS = DIM // LANES


def _distmult_body(ent_hbm, rel_hbm, h_hbm, r_hbm, t_hbm, out_hbm,
                   hidx, ridx, tidx, h_rows, r_rows, t_rows, q, out_v, sem):
    wid = lax.axis_index("s") * NUM_CORES + lax.axis_index("c")
    base = wid * ROWS_PER_TILE

    pltpu.sync_copy(h_hbm.at[pl.ds(base, ROWS_PER_TILE)], hidx)
    pltpu.sync_copy(r_hbm.at[pl.ds(base, ROWS_PER_TILE)], ridx)
    pltpu.sync_copy(t_hbm.at[pl.ds(base, ROWS_PER_TILE)], tidx)

    copies = []
    for j in range(NCHUNK):
        sl = pl.ds(j * CHUNK, CHUNK)
        copies.append(pltpu.async_copy(ent_hbm.at[hidx.at[sl]], h_rows.at[sl], sem))
        copies.append(pltpu.async_copy(rel_hbm.at[ridx.at[sl]], r_rows.at[sl], sem))
        copies.append(pltpu.async_copy(ent_hbm.at[tidx.at[sl]], t_rows.at[sl], sem))
    for cp in copies:
        cp.wait()

    @pl.loop(0, ROWS_PER_TILE)
    def _(i):
        acc = (h_rows[i, pl.ds(0, LANES)]
               * r_rows[i, pl.ds(0, LANES)]
               * t_rows[i, pl.ds(0, LANES)])
        for c in range(1, DIM_VREGS):
            acc = acc + (h_rows[i, pl.ds(c * LANES, LANES)]
                         * r_rows[i, pl.ds(c * LANES, LANES)]
                         * t_rows[i, pl.ds(c * LANES, LANES)])
        q[i, :] = acc

    lanes_iota = lax.iota(jnp.int32, LANES)

    @pl.loop(0, ROWS_PER_TILE, step=LANES)
    def _(i0):
        rows16 = i0 + lanes_iota
        acc = plsc.load_gather(q, [rows16, jnp.zeros((LANES,), jnp.int32)])
        for l in range(1, LANES):
            acc = acc + plsc.load_gather(q, [rows16, jnp.full((LANES,), l, jnp.int32)])
        out_v[pl.ds(i0, LANES)] = acc

    pltpu.sync_copy(out_v, out_hbm.at[pl.ds(base, ROWS_PER_TILE)])


@jax.jit
def kernel(entity_embeddings, relation_embeddings, h, r, t):
    mesh = plsc.VectorSubcoreMesh(core_axis_name="c", subcore_axis_name="s")
    cp = pltpu.CompilerParams()
    if "needs_layout_passes" in pltpu.CompilerParams.__dataclass_fields__:
        cp = dataclasses.replace(cp, needs_layout_passes=False)
    if "use_tc_tiling_on_sc" in pltpu.CompilerParams.__dataclass_fields__:
        cp = dataclasses.replace(cp, use_tc_tiling_on_sc=False)
    run = pl.kernel(
        _distmult_body,
        out_type=jax.ShapeDtypeStruct((BATCH,), jnp.float32),
        mesh=mesh,
        scratch_types=[
            pltpu.VMEM((ROWS_PER_TILE,), jnp.int32),
            pltpu.VMEM((ROWS_PER_TILE,), jnp.int32),
            pltpu.VMEM((ROWS_PER_TILE,), jnp.int32),
            pltpu.VMEM((ROWS_PER_TILE, DIM), jnp.float32),
            pltpu.VMEM((ROWS_PER_TILE, DIM), jnp.float32),
            pltpu.VMEM((ROWS_PER_TILE, DIM), jnp.float32),
            pltpu.VMEM((ROWS_PER_TILE, LANES), jnp.float32),
            pltpu.VMEM((ROWS_PER_TILE,), jnp.float32),
            pltpu.SemaphoreType.DMA,
        ],
        compiler_params=cp,
    )
    return run(entity_embeddings, relation_embeddings,
               h.astype(jnp.int32), r.astype(jnp.int32), t.astype(jnp.int32))


# speedup vs baseline: 1.1145x; 1.1145x over previous
import dataclasses
import functools

import jax
import jax.numpy as jnp
from jax import lax
from jax.experimental import pallas as pl
from jax.experimental.pallas import tpu as pltpu
from jax.experimental.pallas import tpu_sc as plsc

NUM_CORES = 2
NUM_SUBCORES = 16
NUM_TILES = NUM_CORES * NUM_SUBCORES
LANES = 16

BATCH = 16384
DIM = 64
PAD_DIM = 128
ROWS_PER_TILE = BATCH // NUM_TILES
CHUNK = 128
NCHUNK = ROWS_PER_TILE // CHUNK
DIM_VREGS = DIM // LANES


def _distmult_body(ent_hbm, rel_hbm, h_hbm, r_hbm, t_hbm, out_hbm,
                   hidx, ridx, tidx, h_bufs, r_bufs, t_bufs, q, out_v,
                   sem0, sem1):
    wid = lax.axis_index("s") * NUM_CORES + lax.axis_index("c")
    base = wid * ROWS_PER_TILE

    pltpu.sync_copy(h_hbm.at[pl.ds(base, ROWS_PER_TILE)], hidx)
    pltpu.sync_copy(r_hbm.at[pl.ds(base, ROWS_PER_TILE)], ridx)
    pltpu.sync_copy(t_hbm.at[pl.ds(base, ROWS_PER_TILE)], tidx)

    sems = (sem0, sem1)

    def issue(c):
        par = c % 2
        sl = pl.ds(c * CHUNK, CHUNK)
        return [
            pltpu.async_copy(ent_hbm.at[hidx.at[sl]], h_bufs.at[par], sems[par]),
            pltpu.async_copy(ent_hbm.at[tidx.at[sl]], t_bufs.at[par], sems[par]),
            pltpu.async_copy(rel_hbm.at[ridx.at[sl]], r_bufs.at[par], sems[par]),
        ]

    pending = issue(0)
    for c in range(NCHUNK):
        current = pending
        if c + 1 < NCHUNK:
            pending = issue(c + 1)
        for cp in current:
            cp.wait()
        par = c % 2
        hb, rb, tb = h_bufs.at[par], r_bufs.at[par], t_bufs.at[par]

        @pl.loop(0, CHUNK)
        def _(i):
            acc = (hb[i, pl.ds(0, LANES)]
                   * rb[i, pl.ds(0, LANES)]
                   * tb[i, pl.ds(0, LANES)])
            for d in range(1, DIM_VREGS):
                acc = acc + (hb[i, pl.ds(d * LANES, LANES)]
                             * rb[i, pl.ds(d * LANES, LANES)]
                             * tb[i, pl.ds(d * LANES, LANES)])
            q[c * CHUNK + i, :] = acc

    lanes_iota = lax.iota(jnp.int32, LANES)

    @pl.loop(0, ROWS_PER_TILE, step=LANES)
    def _(i0):
        rows16 = i0 + lanes_iota
        acc = plsc.load_gather(q, [rows16, jnp.zeros((LANES,), jnp.int32)])
        for l in range(1, LANES):
            acc = acc + plsc.load_gather(q, [rows16, jnp.full((LANES,), l, jnp.int32)])
        out_v[pl.ds(i0, LANES)] = acc

    pltpu.sync_copy(out_v, out_hbm.at[pl.ds(base, ROWS_PER_TILE)])


@jax.jit
def kernel(entity_embeddings, relation_embeddings, h, r, t):
    ent_pad = jnp.pad(entity_embeddings, ((0, 0), (0, PAD_DIM - DIM)))
    mesh = plsc.VectorSubcoreMesh(core_axis_name="c", subcore_axis_name="s")
    cp = pltpu.CompilerParams()
    if "needs_layout_passes" in pltpu.CompilerParams.__dataclass_fields__:
        cp = dataclasses.replace(cp, needs_layout_passes=False)
    if "use_tc_tiling_on_sc" in pltpu.CompilerParams.__dataclass_fields__:
        cp = dataclasses.replace(cp, use_tc_tiling_on_sc=False)
    run = pl.kernel(
        _distmult_body,
        out_type=jax.ShapeDtypeStruct((BATCH,), jnp.float32),
        mesh=mesh,
        scratch_types=[
            pltpu.VMEM((ROWS_PER_TILE,), jnp.int32),
            pltpu.VMEM((ROWS_PER_TILE,), jnp.int32),
            pltpu.VMEM((ROWS_PER_TILE,), jnp.int32),
            pltpu.VMEM((2, CHUNK, PAD_DIM), jnp.float32),
            pltpu.VMEM((2, CHUNK, DIM), jnp.float32),
            pltpu.VMEM((2, CHUNK, PAD_DIM), jnp.float32),
            pltpu.VMEM((ROWS_PER_TILE, LANES), jnp.float32),
            pltpu.VMEM((ROWS_PER_TILE,), jnp.float32),
            pltpu.SemaphoreType.DMA,
            pltpu.SemaphoreType.DMA,
        ],
        compiler_params=cp,
    )
    return run(ent_pad, relation_embeddings,
               h.astype(jnp.int32), r.astype(jnp.int32), t.astype(jnp.int32))
